# baseline (speedup 1.0000x reference)
import jax
import jax.numpy as jnp
from jax.experimental import pallas as pl
from jax.experimental.pallas import tpu as pltpu


def _copy_body(ent_ref, rel_ref, ent_out_ref, rel_out_ref, sem_ent, sem_rel):
    ent_copy = pltpu.make_async_copy(ent_ref, ent_out_ref, sem_ent)
    rel_copy = pltpu.make_async_copy(rel_ref, rel_out_ref, sem_rel)
    ent_copy.start()
    rel_copy.start()
    ent_copy.wait()
    rel_copy.wait()


def kernel(x_dict, edge_index, entity_emb, rel_emb):
    ent_out, rel_out = pl.pallas_call(
        _copy_body,
        out_shape=(
            jax.ShapeDtypeStruct(entity_emb.shape, entity_emb.dtype),
            jax.ShapeDtypeStruct(rel_emb.shape, rel_emb.dtype),
        ),
        in_specs=[
            pl.BlockSpec(memory_space=pl.ANY),
            pl.BlockSpec(memory_space=pl.ANY),
        ],
        out_specs=(
            pl.BlockSpec(memory_space=pl.ANY),
            pl.BlockSpec(memory_space=pl.ANY),
        ),
        scratch_shapes=[pltpu.SemaphoreType.DMA, pltpu.SemaphoreType.DMA],
    )(entity_emb, rel_emb)
    return (ent_out, rel_out)


# speedup vs baseline: 16.1076x; 16.1076x over previous
import jax
import jax.numpy as jnp
from jax.experimental import pallas as pl
from jax.experimental.pallas import tpu as pltpu


def _copy_body(x_ref, o_ref):
    o_ref[...] = x_ref[...]


def _grid_copy(x, block_rows):
    n, c = x.shape
    grid = (n + block_rows - 1) // block_rows
    return pl.pallas_call(
        _copy_body,
        out_shape=jax.ShapeDtypeStruct((n, c), x.dtype),
        grid=(grid,),
        in_specs=[pl.BlockSpec((block_rows, c), lambda i: (i, 0))],
        out_specs=pl.BlockSpec((block_rows, c), lambda i: (i, 0)),
    )(x)


def kernel(x_dict, edge_index, entity_emb, rel_emb):
    ent_out = _grid_copy(entity_emb, 8000)
    rel_out = _grid_copy(rel_emb, rel_emb.shape[0])
    return (ent_out, rel_out)


# speedup vs baseline: 16.1105x; 1.0002x over previous
import jax
import jax.numpy as jnp
from jax.experimental import pallas as pl
from jax.experimental.pallas import tpu as pltpu


def _copy_body(x_ref, o_ref):
    o_ref[...] = x_ref[...]


def _grid_copy(x, block_rows):
    n, c = x.shape
    grid = (n + block_rows - 1) // block_rows
    return pl.pallas_call(
        _copy_body,
        out_shape=jax.ShapeDtypeStruct((n, c), x.dtype),
        grid=(grid,),
        in_specs=[pl.BlockSpec((block_rows, c), lambda i: (i, 0))],
        out_specs=pl.BlockSpec((block_rows, c), lambda i: (i, 0)),
        compiler_params=pltpu.CompilerParams(
            dimension_semantics=("parallel",),
        ),
    )(x)


def kernel(x_dict, edge_index, entity_emb, rel_emb):
    ent_out = _grid_copy(entity_emb, 8000)
    rel_out = _grid_copy(rel_emb, rel_emb.shape[0])
    return (ent_out, rel_out)
